# baseline (speedup 1.0000x reference)
import functools
import jax
import jax.numpy as jnp
from jax import lax
from jax.experimental import pallas as pl
from jax.experimental.pallas import tpu as pltpu
from jax.experimental.pallas import tpu_sc as plsc

N = 10000
E = 320000
D = 128
NPAD = 10240
NT = 16
EC = E // NT
STRIPE = NPAD // NT
F32 = jnp.float32
I32 = jnp.int32


def _proj_body(x_ref, flog_ref, pw_ref, pb_ref, w1_ref, elog_ref,
               q_ref, ew_ref):
    xb = x_ref[:]
    i = pl.program_id(0)
    fg = 1.0 / (1.0 + jnp.exp(-flog_ref[:]))
    rows = lax.broadcasted_iota(I32, xb.shape, 0)
    sel = jnp.logical_and(rows == 0, i == 0)
    xb = xb * jnp.where(sel, fg, 1.0)
    p = jnp.maximum(jnp.dot(xb, pw_ref[:], preferred_element_type=F32,
                            precision=lax.Precision.HIGHEST)
                    + pb_ref[:], 0.0)
    q_ref[:] = jnp.dot(p, w1_ref[:], preferred_element_type=F32,
                       precision=lax.Precision.HIGHEST)
    ew_ref[:] = jax.nn.sigmoid(elog_ref[:])


EP = 2560


def _compute_q(x, feat_logits, proj_W, proj_b, gcn_W1, elog):
    blk = N // 10
    eblk = EP // 10
    q, ew = pl.pallas_call(
        _proj_body,
        grid=(10,),
        in_specs=[
            pl.BlockSpec((blk, D), lambda i: (i, 0)),
            pl.BlockSpec((1, D), lambda i: (0, 0)),
            pl.BlockSpec((D, D), lambda i: (0, 0)),
            pl.BlockSpec((1, D), lambda i: (0, 0)),
            pl.BlockSpec((D, D), lambda i: (0, 0)),
            pl.BlockSpec((eblk, D), lambda i: (i, 0)),
        ],
        out_specs=[pl.BlockSpec((blk, D), lambda i: (i, 0)),
                   pl.BlockSpec((eblk, D), lambda i: (i, 0))],
        out_shape=[jax.ShapeDtypeStruct((N, D), F32),
                   jax.ShapeDtypeStruct((EP, D), F32)],
    )(x, feat_logits.reshape(1, D), proj_W, proj_b.reshape(1, D), gcn_W1,
      jnp.pad(elog, (0, EP * D - E)).reshape(EP, D))
    return q, ew.reshape(EP * D)[:E]


def _rsqrt_sc(d):
    xi = plsc.bitcast(d, I32)
    y = plsc.bitcast(jnp.int32(0x5F3759DF) - lax.shift_right_logical(xi, 1), F32)
    for _ in range(3):
        y = y * (1.5 - 0.5 * d * y * y)
    return y


SUB = 10000
CAP = 1536


def _sc_body(src_hbm, dst_hbm, elog_hbm, q_hbm, b1_hbm, su_out, h1t_out,
             src_v, dst_v, w_v, dinv_v, wsum_v, slot_v, red_v, tmp_v,
             ss_v, nl_v, cl_v, rows_v, zrows_v, qrows_v, mrows_v,
             b1_v, acc_v, h1t_v, lane_v, lane2_v, li_v, wl_v,
             degstage, wsumstage, dinv_sh, wsum_sh, slot_sh, m_sh,
             counts_sh, ustage, sem):
    tid = lax.axis_index("s")
    iota16 = lax.broadcasted_iota(I32, (16,), 0)
    zs16 = jnp.zeros((16,), F32)
    zi16 = jnp.zeros((16,), I32)
    nbase = tid * STRIPE

    pltpu.sync_copy(b1_hbm, b1_v)

    def _zero(i, _):
        dinv_v[pl.ds(i * 16, 16)] = zs16
        wsum_v[pl.ds(i * 16, 16)] = zs16
        return 0
    lax.fori_loop(0, NPAD // 16, _zero, 0)

    def _zero2(i, _):
        nl_v[pl.ds(i * 16, 16)] = zi16
        ss_v[pl.ds(i * 16, 16)] = jnp.full((16,), -1, I32)
        return 0
    lax.fori_loop(0, STRIPE // 16, _zero2, 0)
    for i in range(16):
        for c in range(8):
            zrows_v[i, pl.ds(c * 16, 16)] = zs16
    def _zacc(c, _):
        acc_v[pl.ds(c * 16, 16)] = zs16
        h1t_v[pl.ds(c * 16, 16)] = zs16
        return 0
    lax.fori_loop(0, 8, _zacc, 0)

    for s in range(EC // SUB):
        ebase = tid * EC + s * SUB
        c1 = pltpu.async_copy(src_hbm.at[pl.ds(ebase, SUB)], src_v, sem)
        c2 = pltpu.async_copy(dst_hbm.at[pl.ds(ebase, SUB)], dst_v, sem)
        c3 = pltpu.async_copy(elog_hbm.at[pl.ds(ebase, SUB)], w_v, sem)
        c1.wait(); c2.wait(); c3.wait()

        def _scan1(g, _):
            d16 = dst_v[pl.ds(g * 16, 16)]
            s16 = src_v[pl.ds(g * 16, 16)]
            w16 = w_v[pl.ds(g * 16, 16)]
            plsc.addupdate_scatter(dinv_v, [d16], w16)
            plsc.addupdate_scatter(wsum_v, [s16], w16, mask=d16 == 0)
            return 0
        lax.fori_loop(0, SUB // 16, _scan1, 0)

    pltpu.sync_copy(dinv_v, degstage.at[tid])
    pltpu.sync_copy(wsum_v, wsumstage.at[tid])
    plsc.subcore_barrier()

    for stage_sh, out_sh, is_deg in ((degstage, dinv_sh, True),
                                     (wsumstage, wsum_sh, False)):
        pltpu.sync_copy(stage_sh.at[:, pl.ds(nbase, STRIPE)], red_v)

        def _red(c, _):
            acc = red_v[0, pl.ds(c * 16, 16)]
            for r in range(1, NT):
                acc = acc + red_v[r, pl.ds(c * 16, 16)]
            if is_deg:
                tmp_v[pl.ds(c * 16, 16)] = _rsqrt_sc(acc + 1.0)
            else:
                tmp_v[pl.ds(c * 16, 16)] = acc
            return 0
        lax.fori_loop(0, STRIPE // 16, _red, 0)
        pltpu.sync_copy(tmp_v, out_sh.at[pl.ds(nbase, STRIPE)])
    plsc.subcore_barrier()

    pltpu.sync_copy(dinv_sh, dinv_v)
    pltpu.sync_copy(wsum_sh, wsum_v)

    def _assign(k, cnt):
        j16 = nbase + k * 16 + iota16
        ws16 = wsum_v[pl.ds(nbase + k * 16, 16)]
        mask = jnp.logical_or(ws16 > 0.0, j16 == 0)
        inc = plsc.cumsum(jnp.where(mask, 1, 0))
        slotl = cnt + inc - 1
        plsc.store_scatter(nl_v, [jnp.where(mask, slotl, STRIPE - 1)],
                           j16, mask=mask)
        plsc.store_scatter(ss_v, [k * 16 + iota16], slotl, mask=mask)
        return cnt + jnp.max(inc)
    cnt = lax.fori_loop(0, STRIPE // 16, _assign, jnp.int32(0))
    cntp = jnp.bitwise_and(cnt + 15, jnp.int32(-16))
    li_v[...] = jnp.broadcast_to(cntp, (16,))
    pltpu.sync_copy(li_v, counts_sh.at[tid])
    plsc.subcore_barrier()

    pltpu.sync_copy(counts_sh, cl_v)
    off = jnp.int32(0)
    total = jnp.int32(0)
    for r in range(NT):
        c_r = jnp.max(cl_v[r, pl.ds(0, 16)])
        total = total + c_r
        off = off + jnp.where(r < tid, c_r, 0)

    def _pub(k, _):
        sv = ss_v[pl.ds(k * 16, 16)]
        ss_v[pl.ds(k * 16, 16)] = jnp.where(sv >= 0, sv + off, -1)
        return 0
    lax.fori_loop(0, STRIPE // 16, _pub, 0)
    pltpu.sync_copy(ss_v, slot_sh.at[pl.ds(nbase, STRIPE)])
    plsc.subcore_barrier()
    pltpu.sync_copy(slot_sh, slot_v)

    dv0 = dinv_v[pl.ds(0, 16)]
    lane_v[pl.ds(0, 16)] = dv0
    lane_v[pl.ds(16, 16)] = dv0
    dvT = plsc.load_gather(lane_v, [jnp.full((16,), 16, I32)])

    nrounds = lax.div(total + (CAP - 1), jnp.int32(CAP))

    def _round(R, _):
        for z in range(CAP // NT // 16):
            pltpu.sync_copy(
                zrows_v, m_sh.at[pl.ds(tid * (CAP // NT) + z * 16, 16)])
        plsc.subcore_barrier()

        for s in range(EC // SUB):
            ebase = tid * EC + s * SUB
            c1 = pltpu.async_copy(src_hbm.at[pl.ds(ebase, SUB)], src_v, sem)
            c2 = pltpu.async_copy(dst_hbm.at[pl.ds(ebase, SUB)], dst_v, sem)
            c3 = pltpu.async_copy(elog_hbm.at[pl.ds(ebase, SUB)], w_v, sem)
            c1.wait(); c2.wait(); c3.wait()

            def _scan3(g, off16):
                d16 = dst_v[pl.ds(g * 16, 16)]
                sl16 = plsc.load_gather(slot_v, [d16])
                mask = jnp.logical_and(sl16 >= R * CAP,
                                       sl16 < R * CAP + CAP)
                inc = plsc.cumsum(jnp.where(mask, 1, 0))
                idxm = jnp.where(mask, off16 + inc - 1, 0)
                plsc.store_scatter(wl_v, [idxm], g * 16 + iota16, mask=mask)
                return off16 + plsc.all_reduce_population_count(mask)
            off16 = lax.fori_loop(0, SUB // 16, _scan3,
                                  jnp.zeros((16,), I32), unroll=4)
            nw = jnp.max(off16)

            def _proc(k, _):
                lm = (k * 16 + iota16) < nw
                ei16 = wl_v[pl.ds(k * 16, 16)]
                eic = jnp.where(lm, ei16, 0)
                s16 = plsc.load_gather(src_v, [eic])
                d16 = plsc.load_gather(dst_v, [eic])
                w16 = plsc.load_gather(w_v, [eic])
                sl16 = plsc.load_gather(slot_v, [d16])
                idxc = jnp.where(lm, s16, 0)
                slotc = jnp.where(lm, sl16 - R * CAP, 0)
                dvs = plsc.load_gather(dinv_v, [idxc])
                dvd = plsc.load_gather(dinv_v, [jnp.where(lm, d16, 0)])
                norm = jnp.where(lm, dvs * w16 * dvd, 0.0)
                pltpu.async_copy(q_hbm.at[idxc], qrows_v, sem).wait()
                lane_v[pl.ds(0, 16)] = norm
                lane_v[pl.ds(16, 16)] = norm
                for i in range(16):
                    nb = plsc.load_gather(lane_v,
                                          [jnp.full((16,), 16 + i, I32)])
                    for c in range(8):
                        rows_v[i, pl.ds(c * 16, 16)] = (
                            qrows_v[i, pl.ds(c * 16, 16)] * nb)
                pltpu.sync_copy(rows_v, m_sh.at[slotc], add=True)
                return 0
            lax.fori_loop(0, lax.div(nw + 15, jnp.int32(16)), _proc, 0)
        plsc.subcore_barrier()

        def _scan4(k, _):
            gbase = off + k * 16
            active = jnp.logical_and(k * 16 < cnt,
                                     lax.div(gbase, jnp.int32(CAP)) == R)

            @pl.when(active)
            def _():
                j16 = nl_v[pl.ds(k * 16, 16)]
                lanemask = (k * 16 + iota16) < cnt
                pltpu.async_copy(q_hbm.at[j16], qrows_v, sem).wait()
                pltpu.sync_copy(m_sh.at[pl.ds(gbase - R * CAP, 16)], mrows_v)
                dv16 = plsc.load_gather(dinv_v, [j16])
                ws16 = plsc.load_gather(wsum_v, [j16])
                cn16 = jnp.where(lanemask, dv16 * dvT * ws16, 0.0)
                d216 = dv16 * dv16
                lane_v[pl.ds(0, 16)] = cn16
                lane_v[pl.ds(16, 16)] = cn16
                lane2_v[pl.ds(0, 16)] = d216
                lane2_v[pl.ds(16, 16)] = d216
                for i in range(16):
                    ci = jnp.full((16,), 16 + i, I32)
                    cnb = plsc.load_gather(lane_v, [ci])
                    d2b = plsc.load_gather(lane2_v, [ci])
                    for c in range(8):
                        sl = pl.ds(c * 16, 16)
                        h1 = jnp.maximum(
                            b1_v[sl] + mrows_v[i, sl] + d2b * qrows_v[i, sl],
                            0.0)
                        acc_v[sl] = acc_v[sl] + cnb * h1
                        if i == 0:
                            @pl.when(jnp.logical_and(
                                jnp.logical_and(tid == 0, R == 0), k == 0))
                            def _():
                                h1t_v[sl] = h1
            return 0
        lax.fori_loop(0, (STRIPE // 16), _scan4, 0)
        plsc.subcore_barrier()
        return 0
    lax.fori_loop(0, nrounds, _round, 0)

    pltpu.sync_copy(acc_v, ustage.at[tid])
    plsc.subcore_barrier()

    @pl.when(tid == 0)
    def _():
        for r in range(NT):
            pltpu.sync_copy(ustage.at[r], red_v.at[r, pl.ds(0, 128)])
        d2T = dvT * dvT

        def _fin(c, _):
            sl = pl.ds(c * 16, 16)
            acc = red_v[0, sl]
            for r in range(1, NT):
                acc = acc + red_v[r, sl]
            tmp_v[sl] = acc + h1t_v[sl] * d2T
            return 0
        lax.fori_loop(0, 8, _fin, 0)
        pltpu.sync_copy(tmp_v.at[pl.ds(0, 128)], su_out)
        pltpu.sync_copy(h1t_v, h1t_out)


def _sc_sparse(src, dst, elog, qpad, b1):
    mesh = plsc.VectorSubcoreMesh(core_axis_name="c", subcore_axis_name="s",
                                  num_cores=1)
    fn = pl.kernel(
        _sc_body,
        out_type=(jax.ShapeDtypeStruct((D,), F32),
                  jax.ShapeDtypeStruct((D,), F32)),
        mesh=mesh,
        compiler_params=pltpu.CompilerParams(needs_layout_passes=False),
        scratch_types=[
            pltpu.VMEM((SUB,), I32),
            pltpu.VMEM((SUB,), I32),
            pltpu.VMEM((SUB,), F32),
            pltpu.VMEM((NPAD,), F32),
            pltpu.VMEM((NPAD,), F32),
            pltpu.VMEM((NPAD,), I32),
            pltpu.VMEM((NT, STRIPE), F32),
            pltpu.VMEM((STRIPE,), F32),
            pltpu.VMEM((STRIPE,), I32),
            pltpu.VMEM((STRIPE,), I32),
            pltpu.VMEM((NT, 16), I32),
            pltpu.VMEM((16, D), F32),
            pltpu.VMEM((16, D), F32),
            pltpu.VMEM((16, D), F32),
            pltpu.VMEM((16, D), F32),
            pltpu.VMEM((D,), F32),
            pltpu.VMEM((D,), F32),
            pltpu.VMEM((D,), F32),
            pltpu.VMEM((32,), F32),
            pltpu.VMEM((32,), F32),
            pltpu.VMEM((16,), I32),
            pltpu.VMEM((SUB,), I32),
            pltpu.VMEM_SHARED((NT, NPAD), F32),
            pltpu.VMEM_SHARED((NT, NPAD), F32),
            pltpu.VMEM_SHARED((NPAD,), F32),
            pltpu.VMEM_SHARED((NPAD,), F32),
            pltpu.VMEM_SHARED((NPAD,), I32),
            pltpu.VMEM_SHARED((CAP, D), F32),
            pltpu.VMEM_SHARED((NT, 16), I32),
            pltpu.VMEM_SHARED((NT, D), F32),
            pltpu.SemaphoreType.DMA,
        ],
    )
    return fn(src, dst, elog, qpad, b1)


def _head_body(su_ref, h1t_ref, w2_ref, b2_ref, cg_ref, wih_ref, whh_ref,
               bih_ref, bhh_ref, aw_ref, ab_ref, p1_ref, pb1_ref, p2_ref,
               pb2_ref, out_ref):
    h2t = jnp.maximum(jnp.dot(su_ref[:], w2_ref[:], preferred_element_type=F32, precision=lax.Precision.HIGHEST)
                      + b2_ref[:], 0.0)
    emb = jnp.concatenate([h1t_ref[:], h2t], axis=1)
    xs = [cg_ref[0:1, :], cg_ref[1:2, :], emb]
    h = jnp.zeros((1, 128), F32)
    c = jnp.zeros((1, 128), F32)
    hs = []
    for t in range(3):
        g = (jnp.dot(xs[t], wih_ref[:], preferred_element_type=F32, precision=lax.Precision.HIGHEST)
             + bih_ref[:]
             + jnp.dot(h, whh_ref[:], preferred_element_type=F32, precision=lax.Precision.HIGHEST)
             + bhh_ref[:])
        ig = jax.nn.sigmoid(g[:, 0:128])
        fg = jax.nn.sigmoid(g[:, 128:256])
        gg = jnp.tanh(g[:, 256:384])
        og = jax.nn.sigmoid(g[:, 384:512])
        c = fg * c + ig * gg
        h = og * jnp.tanh(c)
        hs.append(h)
    ss = [jnp.tanh(jnp.dot(hh, aw_ref[:], preferred_element_type=F32, precision=lax.Precision.HIGHEST)
                   + ab_ref[:]) for hh in hs]
    mx = jnp.maximum(jnp.maximum(ss[0], ss[1]), ss[2])
    es = [jnp.exp(s - mx) for s in ss]
    z = es[0] + es[1] + es[2]
    ws = [e / z for e in es]
    ctx = ws[0] * hs[0] + ws[1] * hs[1] + ws[2] * hs[2]
    hm = jnp.maximum(jnp.dot(ctx, p1_ref[:], preferred_element_type=F32, precision=lax.Precision.HIGHEST)
                     + pb1_ref[:], 0.0)
    raw = jnp.dot(hm, p2_ref[:], preferred_element_type=F32, precision=lax.Precision.HIGHEST) + pb2_ref[:]
    pred = jnp.maximum(raw, 0.0) + jnp.log(1.0 + jnp.exp(-jnp.abs(raw)))
    vals = jnp.concatenate([pred, ws[0], ws[1], ws[2]], axis=1)
    out_ref[:] = jnp.pad(vals, ((0, 0), (0, 124)))


def _head(su, h1t, gcn_W2, gcn_b2, cached_gcn, W_ih, W_hh, b_ih, b_hh,
          attn_W, attn_b, pred_W1, pred_b1, pred_W2, pred_b2):
    args = (su.reshape(1, D), h1t.reshape(1, D), gcn_W2, gcn_b2.reshape(1, D),
            cached_gcn, W_ih.T, W_hh.T, b_ih.reshape(1, 512),
            b_hh.reshape(1, 512), attn_W, attn_b.reshape(1, 1),
            pred_W1, pred_b1.reshape(1, 64), pred_W2, pred_b2.reshape(1, 1))
    return pl.pallas_call(
        _head_body,
        in_specs=[pl.BlockSpec(a.shape, lambda: tuple(0 for _ in a.shape))
                  for a in args],
        out_specs=pl.BlockSpec((1, 128), lambda: (0, 0)),
        out_shape=jax.ShapeDtypeStruct((1, 128), F32),
    )(*args)


def kernel(x, edge_index, feat_logits, edge_logits, cached_gcn, proj_W,
           proj_b, gcn_W1, gcn_b1, gcn_W2, gcn_b2, W_ih, W_hh, b_ih, b_hh,
           attn_W, attn_b, pred_W1, pred_b1, pred_W2, pred_b2):
    q, ew = _compute_q(x, feat_logits, proj_W, proj_b, gcn_W1, edge_logits)
    src = edge_index[0]
    dst = edge_index[1]
    su, h1t = _sc_sparse(src, dst, ew, q, gcn_b1)
    out = _head(su, h1t, gcn_W2, gcn_b2, cached_gcn, W_ih, W_hh, b_ih, b_hh,
                attn_W, attn_b, pred_W1, pred_b1, pred_W2, pred_b2)
    pred = out[0, 0]
    weights = out[0, 1:4].reshape(3, 1)
    return pred, weights


# speedup vs baseline: 1.0806x; 1.0806x over previous
import functools
import jax
import jax.numpy as jnp
from jax import lax
from jax.experimental import pallas as pl
from jax.experimental.pallas import tpu as pltpu
from jax.experimental.pallas import tpu_sc as plsc

N = 10000
E = 320000
D = 128
NPAD = 10240
NT = 16
EC = E // NT
STRIPE = NPAD // NT
F32 = jnp.float32
I32 = jnp.int32


def _proj_body(x_ref, flog_ref, pw_ref, pb_ref, w1_ref, q_ref):
    xb = x_ref[:]
    i = pl.program_id(0)
    fg = 1.0 / (1.0 + jnp.exp(-flog_ref[:]))
    rows = lax.broadcasted_iota(I32, xb.shape, 0)
    sel = jnp.logical_and(rows == 0, i == 0)
    xb = xb * jnp.where(sel, fg, 1.0)
    p = jnp.maximum(jnp.dot(xb, pw_ref[:], preferred_element_type=F32,
                            precision=lax.Precision.HIGHEST)
                    + pb_ref[:], 0.0)
    q_ref[:] = jnp.dot(p, w1_ref[:], preferred_element_type=F32,
                       precision=lax.Precision.HIGHEST)


def _compute_q(xpad, feat_logits, proj_W, proj_b, gcn_W1):
    blk = 1024
    return pl.pallas_call(
        _proj_body,
        grid=(NPAD // blk,),
        in_specs=[
            pl.BlockSpec((blk, D), lambda i: (i, 0)),
            pl.BlockSpec((1, D), lambda i: (0, 0)),
            pl.BlockSpec((D, D), lambda i: (0, 0)),
            pl.BlockSpec((1, D), lambda i: (0, 0)),
            pl.BlockSpec((D, D), lambda i: (0, 0)),
        ],
        out_specs=pl.BlockSpec((blk, D), lambda i: (i, 0)),
        out_shape=jax.ShapeDtypeStruct((NPAD, D), F32),
    )(xpad, feat_logits.reshape(1, D), proj_W, proj_b.reshape(1, D), gcn_W1)


def _esig_body(x_ref, o_ref):
    o_ref[:] = jax.nn.sigmoid(x_ref[:])


def _edge_sigmoid(elog):
    return pl.pallas_call(
        _esig_body,
        in_specs=[pl.BlockSpec((E // 128, 128), lambda: (0, 0))],
        out_specs=pl.BlockSpec((E // 128, 128), lambda: (0, 0)),
        out_shape=jax.ShapeDtypeStruct((E // 128, 128), F32),
    )(elog.reshape(E // 128, 128)).reshape(E)


def _rsqrt_sc(d):
    xi = plsc.bitcast(d, I32)
    y = plsc.bitcast(jnp.int32(0x5F3759DF) - lax.shift_right_logical(xi, 1), F32)
    for _ in range(3):
        y = y * (1.5 - 0.5 * d * y * y)
    return y


SUB = 10000
CAP = 1536


def _sc_body(src_hbm, dst_hbm, elog_hbm, q_hbm, b1_hbm, su_out, h1t_out,
             src_v, dst_v, w_v, dinv_v, wsum_v, slot_v, red_v, tmp_v,
             ss_v, nl_v, cl_v, rows_v, zrows_v, qrows_v, mrows_v,
             b1_v, acc_v, h1t_v, lane_v, lane2_v, li_v, wl_v,
             degstage, wsumstage, dinv_sh, wsum_sh, slot_sh, m_sh,
             counts_sh, ustage, sem):
    tid = lax.axis_index("s")
    iota16 = lax.broadcasted_iota(I32, (16,), 0)
    zs16 = jnp.zeros((16,), F32)
    zi16 = jnp.zeros((16,), I32)
    nbase = tid * STRIPE

    pltpu.sync_copy(b1_hbm, b1_v)

    def _zero(i, _):
        dinv_v[pl.ds(i * 16, 16)] = zs16
        wsum_v[pl.ds(i * 16, 16)] = zs16
        return 0
    lax.fori_loop(0, NPAD // 16, _zero, 0)

    def _zero2(i, _):
        nl_v[pl.ds(i * 16, 16)] = zi16
        ss_v[pl.ds(i * 16, 16)] = jnp.full((16,), -1, I32)
        return 0
    lax.fori_loop(0, STRIPE // 16, _zero2, 0)
    for i in range(16):
        for c in range(8):
            zrows_v[i, pl.ds(c * 16, 16)] = zs16
    def _zacc(c, _):
        acc_v[pl.ds(c * 16, 16)] = zs16
        h1t_v[pl.ds(c * 16, 16)] = zs16
        return 0
    lax.fori_loop(0, 8, _zacc, 0)

    for s in range(EC // SUB):
        ebase = tid * EC + s * SUB
        c1 = pltpu.async_copy(src_hbm.at[pl.ds(ebase, SUB)], src_v, sem)
        c2 = pltpu.async_copy(dst_hbm.at[pl.ds(ebase, SUB)], dst_v, sem)
        c3 = pltpu.async_copy(elog_hbm.at[pl.ds(ebase, SUB)], w_v, sem)
        c1.wait(); c2.wait(); c3.wait()

        def _scan1(g, _):
            d16 = dst_v[pl.ds(g * 16, 16)]
            s16 = src_v[pl.ds(g * 16, 16)]
            w16 = w_v[pl.ds(g * 16, 16)]
            plsc.addupdate_scatter(dinv_v, [d16], w16)
            plsc.addupdate_scatter(wsum_v, [s16], w16, mask=d16 == 0)
            return 0
        lax.fori_loop(0, SUB // 16, _scan1, 0)

    pltpu.sync_copy(dinv_v, degstage.at[tid])
    pltpu.sync_copy(wsum_v, wsumstage.at[tid])
    plsc.subcore_barrier()

    for stage_sh, out_sh, is_deg in ((degstage, dinv_sh, True),
                                     (wsumstage, wsum_sh, False)):
        pltpu.sync_copy(stage_sh.at[:, pl.ds(nbase, STRIPE)], red_v)

        def _red(c, _):
            acc = red_v[0, pl.ds(c * 16, 16)]
            for r in range(1, NT):
                acc = acc + red_v[r, pl.ds(c * 16, 16)]
            if is_deg:
                tmp_v[pl.ds(c * 16, 16)] = _rsqrt_sc(acc + 1.0)
            else:
                tmp_v[pl.ds(c * 16, 16)] = acc
            return 0
        lax.fori_loop(0, STRIPE // 16, _red, 0)
        pltpu.sync_copy(tmp_v, out_sh.at[pl.ds(nbase, STRIPE)])
    plsc.subcore_barrier()

    pltpu.sync_copy(dinv_sh, dinv_v)
    pltpu.sync_copy(wsum_sh, wsum_v)

    def _assign(k, cnt):
        j16 = nbase + k * 16 + iota16
        ws16 = wsum_v[pl.ds(nbase + k * 16, 16)]
        mask = jnp.logical_or(ws16 > 0.0, j16 == 0)
        inc = plsc.cumsum(jnp.where(mask, 1, 0))
        slotl = cnt + inc - 1
        plsc.store_scatter(nl_v, [jnp.where(mask, slotl, STRIPE - 1)],
                           j16, mask=mask)
        plsc.store_scatter(ss_v, [k * 16 + iota16], slotl, mask=mask)
        return cnt + jnp.max(inc)
    cnt = lax.fori_loop(0, STRIPE // 16, _assign, jnp.int32(0))
    cntp = jnp.bitwise_and(cnt + 15, jnp.int32(-16))
    li_v[...] = jnp.broadcast_to(cntp, (16,))
    pltpu.sync_copy(li_v, counts_sh.at[tid])
    plsc.subcore_barrier()

    pltpu.sync_copy(counts_sh, cl_v)
    off = jnp.int32(0)
    total = jnp.int32(0)
    for r in range(NT):
        c_r = jnp.max(cl_v[r, pl.ds(0, 16)])
        total = total + c_r
        off = off + jnp.where(r < tid, c_r, 0)

    def _pub(k, _):
        sv = ss_v[pl.ds(k * 16, 16)]
        ss_v[pl.ds(k * 16, 16)] = jnp.where(sv >= 0, sv + off, -1)
        return 0
    lax.fori_loop(0, STRIPE // 16, _pub, 0)
    pltpu.sync_copy(ss_v, slot_sh.at[pl.ds(nbase, STRIPE)])
    plsc.subcore_barrier()
    pltpu.sync_copy(slot_sh, slot_v)

    dv0 = dinv_v[pl.ds(0, 16)]
    lane_v[pl.ds(0, 16)] = dv0
    lane_v[pl.ds(16, 16)] = dv0
    dvT = plsc.load_gather(lane_v, [jnp.full((16,), 16, I32)])

    nrounds = lax.div(total + (CAP - 1), jnp.int32(CAP))

    def _round(R, _):
        for z in range(CAP // NT // 16):
            pltpu.sync_copy(
                zrows_v, m_sh.at[pl.ds(tid * (CAP // NT) + z * 16, 16)])
        plsc.subcore_barrier()

        for s in range(EC // SUB):
            ebase = tid * EC + s * SUB
            c1 = pltpu.async_copy(src_hbm.at[pl.ds(ebase, SUB)], src_v, sem)
            c2 = pltpu.async_copy(dst_hbm.at[pl.ds(ebase, SUB)], dst_v, sem)
            c3 = pltpu.async_copy(elog_hbm.at[pl.ds(ebase, SUB)], w_v, sem)
            c1.wait(); c2.wait(); c3.wait()

            def _scan3(g, off16):
                d16 = dst_v[pl.ds(g * 16, 16)]
                sl16 = plsc.load_gather(slot_v, [d16])
                mask = jnp.logical_and(sl16 >= R * CAP,
                                       sl16 < R * CAP + CAP)
                inc = plsc.cumsum(jnp.where(mask, 1, 0))
                idxm = jnp.where(mask, off16 + inc - 1, 0)
                plsc.store_scatter(wl_v, [idxm], g * 16 + iota16, mask=mask)
                return off16 + plsc.all_reduce_population_count(mask)
            off16 = lax.fori_loop(0, SUB // 16, _scan3,
                                  jnp.zeros((16,), I32), unroll=4)
            nw = jnp.max(off16)

            def _proc(k, _):
                lm = (k * 16 + iota16) < nw
                ei16 = wl_v[pl.ds(k * 16, 16)]
                eic = jnp.where(lm, ei16, 0)
                s16 = plsc.load_gather(src_v, [eic])
                d16 = plsc.load_gather(dst_v, [eic])
                w16 = plsc.load_gather(w_v, [eic])
                sl16 = plsc.load_gather(slot_v, [d16])
                idxc = jnp.where(lm, s16, 0)
                slotc = jnp.where(lm, sl16 - R * CAP, 0)
                dvs = plsc.load_gather(dinv_v, [idxc])
                dvd = plsc.load_gather(dinv_v, [jnp.where(lm, d16, 0)])
                norm = jnp.where(lm, dvs * w16 * dvd, 0.0)
                pltpu.async_copy(q_hbm.at[idxc], qrows_v, sem).wait()
                lane_v[pl.ds(0, 16)] = norm
                lane_v[pl.ds(16, 16)] = norm
                for i in range(16):
                    nb = plsc.load_gather(lane_v,
                                          [jnp.full((16,), 16 + i, I32)])
                    for c in range(8):
                        rows_v[i, pl.ds(c * 16, 16)] = (
                            qrows_v[i, pl.ds(c * 16, 16)] * nb)
                pltpu.sync_copy(rows_v, m_sh.at[slotc], add=True)
                return 0
            lax.fori_loop(0, lax.div(nw + 15, jnp.int32(16)), _proc, 0)
        plsc.subcore_barrier()

        def _scan4(k, _):
            gbase = off + k * 16
            active = jnp.logical_and(k * 16 < cnt,
                                     lax.div(gbase, jnp.int32(CAP)) == R)

            @pl.when(active)
            def _():
                j16 = nl_v[pl.ds(k * 16, 16)]
                lanemask = (k * 16 + iota16) < cnt
                pltpu.async_copy(q_hbm.at[j16], qrows_v, sem).wait()
                pltpu.sync_copy(m_sh.at[pl.ds(gbase - R * CAP, 16)], mrows_v)
                dv16 = plsc.load_gather(dinv_v, [j16])
                ws16 = plsc.load_gather(wsum_v, [j16])
                cn16 = jnp.where(lanemask, dv16 * dvT * ws16, 0.0)
                d216 = dv16 * dv16
                lane_v[pl.ds(0, 16)] = cn16
                lane_v[pl.ds(16, 16)] = cn16
                lane2_v[pl.ds(0, 16)] = d216
                lane2_v[pl.ds(16, 16)] = d216
                for i in range(16):
                    ci = jnp.full((16,), 16 + i, I32)
                    cnb = plsc.load_gather(lane_v, [ci])
                    d2b = plsc.load_gather(lane2_v, [ci])
                    for c in range(8):
                        sl = pl.ds(c * 16, 16)
                        h1 = jnp.maximum(
                            b1_v[sl] + mrows_v[i, sl] + d2b * qrows_v[i, sl],
                            0.0)
                        acc_v[sl] = acc_v[sl] + cnb * h1
                        if i == 0:
                            @pl.when(jnp.logical_and(
                                jnp.logical_and(tid == 0, R == 0), k == 0))
                            def _():
                                h1t_v[sl] = h1
            return 0
        lax.fori_loop(0, (STRIPE // 16), _scan4, 0)
        plsc.subcore_barrier()
        return 0
    lax.fori_loop(0, nrounds, _round, 0)

    pltpu.sync_copy(acc_v, ustage.at[tid])
    plsc.subcore_barrier()

    @pl.when(tid == 0)
    def _():
        for r in range(NT):
            pltpu.sync_copy(ustage.at[r], red_v.at[r, pl.ds(0, 128)])
        d2T = dvT * dvT

        def _fin(c, _):
            sl = pl.ds(c * 16, 16)
            acc = red_v[0, sl]
            for r in range(1, NT):
                acc = acc + red_v[r, sl]
            tmp_v[sl] = acc + h1t_v[sl] * d2T
            return 0
        lax.fori_loop(0, 8, _fin, 0)
        pltpu.sync_copy(tmp_v.at[pl.ds(0, 128)], su_out)
        pltpu.sync_copy(h1t_v, h1t_out)


def _sc_sparse(src, dst, elog, qpad, b1):
    mesh = plsc.VectorSubcoreMesh(core_axis_name="c", subcore_axis_name="s",
                                  num_cores=1)
    fn = pl.kernel(
        _sc_body,
        out_type=(jax.ShapeDtypeStruct((D,), F32),
                  jax.ShapeDtypeStruct((D,), F32)),
        mesh=mesh,
        compiler_params=pltpu.CompilerParams(needs_layout_passes=False),
        scratch_types=[
            pltpu.VMEM((SUB,), I32),
            pltpu.VMEM((SUB,), I32),
            pltpu.VMEM((SUB,), F32),
            pltpu.VMEM((NPAD,), F32),
            pltpu.VMEM((NPAD,), F32),
            pltpu.VMEM((NPAD,), I32),
            pltpu.VMEM((NT, STRIPE), F32),
            pltpu.VMEM((STRIPE,), F32),
            pltpu.VMEM((STRIPE,), I32),
            pltpu.VMEM((STRIPE,), I32),
            pltpu.VMEM((NT, 16), I32),
            pltpu.VMEM((16, D), F32),
            pltpu.VMEM((16, D), F32),
            pltpu.VMEM((16, D), F32),
            pltpu.VMEM((16, D), F32),
            pltpu.VMEM((D,), F32),
            pltpu.VMEM((D,), F32),
            pltpu.VMEM((D,), F32),
            pltpu.VMEM((32,), F32),
            pltpu.VMEM((32,), F32),
            pltpu.VMEM((16,), I32),
            pltpu.VMEM((SUB,), I32),
            pltpu.VMEM_SHARED((NT, NPAD), F32),
            pltpu.VMEM_SHARED((NT, NPAD), F32),
            pltpu.VMEM_SHARED((NPAD,), F32),
            pltpu.VMEM_SHARED((NPAD,), F32),
            pltpu.VMEM_SHARED((NPAD,), I32),
            pltpu.VMEM_SHARED((CAP, D), F32),
            pltpu.VMEM_SHARED((NT, 16), I32),
            pltpu.VMEM_SHARED((NT, D), F32),
            pltpu.SemaphoreType.DMA,
        ],
    )
    return fn(src, dst, elog, qpad, b1)


def _head_body(su_ref, h1t_ref, w2_ref, b2_ref, cg_ref, wih_ref, whh_ref,
               bih_ref, bhh_ref, aw_ref, ab_ref, p1_ref, pb1_ref, p2_ref,
               pb2_ref, out_ref):
    h2t = jnp.maximum(jnp.dot(su_ref[:], w2_ref[:], preferred_element_type=F32, precision=lax.Precision.HIGHEST)
                      + b2_ref[:], 0.0)
    emb = jnp.concatenate([h1t_ref[:], h2t], axis=1)
    xs = [cg_ref[0:1, :], cg_ref[1:2, :], emb]
    h = jnp.zeros((1, 128), F32)
    c = jnp.zeros((1, 128), F32)
    hs = []
    for t in range(3):
        g = (jnp.dot(xs[t], wih_ref[:], preferred_element_type=F32, precision=lax.Precision.HIGHEST)
             + bih_ref[:]
             + jnp.dot(h, whh_ref[:], preferred_element_type=F32, precision=lax.Precision.HIGHEST)
             + bhh_ref[:])
        ig = jax.nn.sigmoid(g[:, 0:128])
        fg = jax.nn.sigmoid(g[:, 128:256])
        gg = jnp.tanh(g[:, 256:384])
        og = jax.nn.sigmoid(g[:, 384:512])
        c = fg * c + ig * gg
        h = og * jnp.tanh(c)
        hs.append(h)
    ss = [jnp.tanh(jnp.dot(hh, aw_ref[:], preferred_element_type=F32, precision=lax.Precision.HIGHEST)
                   + ab_ref[:]) for hh in hs]
    mx = jnp.maximum(jnp.maximum(ss[0], ss[1]), ss[2])
    es = [jnp.exp(s - mx) for s in ss]
    z = es[0] + es[1] + es[2]
    ws = [e / z for e in es]
    ctx = ws[0] * hs[0] + ws[1] * hs[1] + ws[2] * hs[2]
    hm = jnp.maximum(jnp.dot(ctx, p1_ref[:], preferred_element_type=F32, precision=lax.Precision.HIGHEST)
                     + pb1_ref[:], 0.0)
    raw = jnp.dot(hm, p2_ref[:], preferred_element_type=F32, precision=lax.Precision.HIGHEST) + pb2_ref[:]
    pred = jnp.maximum(raw, 0.0) + jnp.log(1.0 + jnp.exp(-jnp.abs(raw)))
    vals = jnp.concatenate([pred, ws[0], ws[1], ws[2]], axis=1)
    out_ref[:] = jnp.pad(vals, ((0, 0), (0, 124)))


def _head(su, h1t, gcn_W2, gcn_b2, cached_gcn, W_ih, W_hh, b_ih, b_hh,
          attn_W, attn_b, pred_W1, pred_b1, pred_W2, pred_b2):
    args = (su.reshape(1, D), h1t.reshape(1, D), gcn_W2, gcn_b2.reshape(1, D),
            cached_gcn, W_ih.T, W_hh.T, b_ih.reshape(1, 512),
            b_hh.reshape(1, 512), attn_W, attn_b.reshape(1, 1),
            pred_W1, pred_b1.reshape(1, 64), pred_W2, pred_b2.reshape(1, 1))
    return pl.pallas_call(
        _head_body,
        in_specs=[pl.BlockSpec(a.shape, lambda: tuple(0 for _ in a.shape))
                  for a in args],
        out_specs=pl.BlockSpec((1, 128), lambda: (0, 0)),
        out_shape=jax.ShapeDtypeStruct((1, 128), F32),
    )(*args)


def kernel(x, edge_index, feat_logits, edge_logits, cached_gcn, proj_W,
           proj_b, gcn_W1, gcn_b1, gcn_W2, gcn_b2, W_ih, W_hh, b_ih, b_hh,
           attn_W, attn_b, pred_W1, pred_b1, pred_W2, pred_b2):
    xpad = jnp.pad(x, ((0, NPAD - N), (0, 0)))
    qpad = _compute_q(xpad, feat_logits, proj_W, proj_b, gcn_W1)
    src = edge_index[0]
    dst = edge_index[1]
    ew = _edge_sigmoid(edge_logits)
    su, h1t = _sc_sparse(src, dst, ew, qpad, gcn_b1)
    out = _head(su, h1t, gcn_W2, gcn_b2, cached_gcn, W_ih, W_hh, b_ih, b_hh,
                attn_W, attn_b, pred_W1, pred_b1, pred_W2, pred_b2)
    pred = out[0, 0]
    weights = out[0, 1:4].reshape(3, 1)
    return pred, weights
